# baseline (speedup 1.0000x reference)
import functools
import math

import jax
import jax.numpy as jnp
from jax import lax
from jax.experimental import pallas as pl
from jax.experimental.pallas import tpu as pltpu
from jax.experimental.pallas import tpu_sc as plsc

D_MODEL = 64
D_PAD = 128
N_WORKERS = 32
SEQ = 200
BATCHES_PER_WORKER = 128
SPLIT = 128
SCALE = math.sqrt(D_MODEL)

_mesh = plsc.VectorSubcoreMesh(core_axis_name="c", subcore_axis_name="s")


@functools.partial(
    pl.kernel,
    mesh=_mesh,
    compiler_params=pltpu.CompilerParams(use_tc_tiling_on_sc=True),
    out_type=jax.ShapeDtypeStruct((N_WORKERS * BATCHES_PER_WORKER, SEQ,
                                   D_MODEL), jnp.float32),
    scratch_types=[
        pltpu.VMEM((BATCHES_PER_WORKER, SEQ), jnp.int32),
        pltpu.VMEM((2, SEQ, D_PAD), jnp.float32),
        pltpu.VMEM((SEQ, D_MODEL), jnp.float32),
        pltpu.SemaphoreType.DMA,
        pltpu.SemaphoreType.DMA,
    ],
)
def _embed_sc(x_hbm, table_hbm, out_hbm, idx_v, rows_v, dense_v, sem0, sem1):
    wid = lax.axis_index("s") * 2 + lax.axis_index("c")
    base = wid * BATCHES_PER_WORKER
    sems = (sem0, sem1)

    pltpu.sync_copy(x_hbm.at[pl.ds(base, BATCHES_PER_WORKER)], idx_v)

    def fire(b, q):
        pltpu.async_copy(
            table_hbm.at[idx_v.at[b, pl.ds(0, SPLIT)]],
            rows_v.at[q, pl.ds(0, SPLIT)], sems[q])
        pltpu.async_copy(
            table_hbm.at[idx_v.at[b, pl.ds(SPLIT, SEQ - SPLIT)]],
            rows_v.at[q, pl.ds(SPLIT, SEQ - SPLIT)], sems[q])

    def drain(q):
        pltpu.make_async_copy(
            table_hbm.at[pl.ds(0, SEQ)], rows_v.at[q], sems[q]).wait()

    fire(0, 0)
    fire(1, 1)

    def outer(b2, carry):
        for q in range(2):
            b = b2 * 2 + q
            drain(q)

            buf = rows_v.at[q]

            @plsc.parallel_loop(0, SEQ, step=8, unroll=2)
            def _scale(i):
                for k in range(8):
                    for j in range(D_MODEL // 16):
                        sl = pl.ds(j * 16, 16)
                        dense_v[i + k, sl] = buf[i + k, sl] * SCALE

            pltpu.sync_copy(dense_v, out_hbm.at[base + b])

            @pl.when(b2 < BATCHES_PER_WORKER // 2 - 1)
            def _():
                fire(b + 2, q)
        return carry

    lax.fori_loop(0, BATCHES_PER_WORKER // 2, outer, 0)


def kernel(x, table):
    table_pad = jnp.pad(table, ((0, 0), (0, D_PAD - D_MODEL)))
    return _embed_sc(x, table_pad)
